# baseline (speedup 1.0000x reference)
import functools

import jax
import jax.numpy as jnp
import numpy as np
from jax import lax
from jax.experimental import pallas as pl
from jax.experimental.pallas import tpu as pltpu
from jax.experimental.pallas import tpu_sc as plsc

_FEAT_STRIDE = 16
_H = 50
_W = 50
_A = 9
_N = _H * _W * _A
_R, _C = 176, 128
_PAD = _R * _C - _N
_PRE_TOPN = 6000
_POST_TOPN = 300
_THRESH = 0.7
_IMIN = -2147483648

_NT = 16
_CH = (_R * _C) // _NT
_VC = _CH // 16
_KR, _KC = 56, 128
_DUMP = -3e38


def _anchor_consts():
    scales = np.array([8.0, 16.0, 32.0])
    ratios = np.array([0.5, 1.0, 2.0])
    base = np.array([1.0, 1.0, 16.0, 16.0]) - 1
    w = base[2] - base[0] + 1
    h = base[3] - base[1] + 1
    x_ctr = base[0] + 0.5 * (w - 1)
    y_ctr = base[1] + 0.5 * (h - 1)
    size_ratios = w * h / ratios
    ws = np.round(np.sqrt(size_ratios))
    hs = np.round(ws * ratios)

    def _mk(ws_, hs_, xc, yc):
        ws_ = ws_[:, None]
        hs_ = hs_[:, None]
        return np.hstack([xc - 0.5 * (ws_ - 1), yc - 0.5 * (hs_ - 1),
                          xc + 0.5 * (ws_ - 1), yc + 0.5 * (hs_ - 1)])

    ratio_anchors = _mk(ws, hs, x_ctr, y_ctr)
    out = []
    for i in range(ratio_anchors.shape[0]):
        a = ratio_anchors[i]
        aw = a[2] - a[0] + 1
        ah = a[3] - a[1] + 1
        axc = a[0] + 0.5 * (aw - 1)
        ayc = a[1] + 0.5 * (ah - 1)
        out.append(_mk(aw * scales, ah * scales, axc, ayc))
    base9 = np.vstack(out).astype(np.float32)

    shift = np.arange(_W, dtype=np.float32) * np.float32(_FEAT_STRIDE)
    sx, sy = np.meshgrid(shift, shift)
    shifts = np.stack([sx.ravel(), sy.ravel(), sx.ravel(), sy.ravel()], axis=1)
    anchors = (shifts[:, None, :].astype(np.float32)
               + base9[None, :, :]).reshape(-1, 4).astype(np.float32)
    x1, y1, x2, y2 = anchors[:, 0], anchors[:, 1], anchors[:, 2], anchors[:, 3]
    wa = x2 - x1 + np.float32(1.0)
    ha = y2 - y1 + np.float32(1.0)
    cxa = x1 + np.float32(0.5) * wa
    cya = y1 + np.float32(0.5) * ha

    def _pad(v):
        return np.pad(v, (0, _PAD)).reshape(_R, _C)

    return _pad(wa), _pad(ha), _pad(cxa), _pad(cya)


_WA, _HA, _CXA, _CYA = _anchor_consts()


def _tree(v, op, rows):
    chunks = [v[8 * i:8 * (i + 1)] for i in range(rows // 8)]
    while len(chunks) > 1:
        nxt = [op(chunks[2 * j], chunks[2 * j + 1])
               for j in range(len(chunks) // 2)]
        if len(chunks) % 2:
            nxt.append(chunks[-1])
        chunks = nxt
    r = chunks[0]
    if op is jnp.minimum:
        return jnp.min(jnp.min(r, axis=0, keepdims=True), axis=1, keepdims=True)
    if op is jnp.add:
        return jnp.sum(jnp.sum(r, axis=0, keepdims=True), axis=1, keepdims=True)
    return jnp.max(jnp.max(r, axis=0, keepdims=True), axis=1, keepdims=True)


def _decode_body(fg_ref, bg_ref, dx_ref, dy_ref, dw_ref, dh_ref,
                 wa_ref, ha_ref, cx_ref, cy_ref,
                 sc_ref, x1_ref, y1_ref, x2_ref, y2_ref, thr_ref):
    fg = fg_ref[...]
    bg = bg_ref[...]
    wa = wa_ref[...]
    ha = ha_ref[...]

    m = jnp.maximum(fg, bg)
    ef = jnp.exp(fg - m)
    eb = jnp.exp(bg - m)
    sc = ef / (eb + ef)

    pcx = dx_ref[...] * wa + cx_ref[...]
    pcy = dy_ref[...] * ha + cy_ref[...]
    pw = jnp.exp(dw_ref[...]) * wa
    ph = jnp.exp(dh_ref[...]) * ha
    x1 = jnp.clip(pcx - 0.5 * pw, 0.0, 799.0)
    y1 = jnp.clip(pcy - 0.5 * ph, 0.0, 799.0)
    x2 = jnp.clip(pcx + 0.5 * pw, 0.0, 799.0)
    y2 = jnp.clip(pcy + 0.5 * ph, 0.0, 799.0)

    ws = x2 - x1 + 1.0
    hs = y2 - y1 + 1.0
    ok = (ws >= 16.0) & (hs >= 16.0)
    sc = jnp.where(ok, sc, jnp.float32(-1e9))

    ri = lax.broadcasted_iota(jnp.int32, (_R, _C), 0)
    ci = lax.broadcasted_iota(jnp.int32, (_R, _C), 1)
    flat = ri * _C + ci
    sc = jnp.where(flat < _N, sc, jnp.float32(-2e9))

    key = lax.bitcast_convert_type(sc, jnp.int32)
    key = key ^ ((key >> 31) & jnp.int32(0x7FFFFFFF))

    def bs(_, lh):
        lo, hi = lh
        mid = (lo >> 1) + (hi >> 1) + ((lo | hi) & 1)
        cnt = _tree((key >= mid).astype(jnp.int32), jnp.add, _R)
        p = cnt >= _PRE_TOPN
        return (jnp.where(p, mid, lo), jnp.where(p, hi, mid - 1))

    thr, _ = lax.fori_loop(
        0, 32, bs,
        (jnp.full((1, 1), _IMIN, jnp.int32),
         jnp.full((1, 1), 2147483647, jnp.int32)))
    ty = jnp.where(thr >= 0, thr, thr ^ jnp.int32(0x7FFFFFFF))
    thrf = lax.bitcast_convert_type(ty, jnp.float32)

    sc_ref[...] = sc
    x1_ref[...] = x1
    y1_ref[...] = y1
    x2_ref[...] = x2
    y2_ref[...] = y2
    thr_ref[...] = jnp.broadcast_to(thrf, (8, 128))


def _compact_body(sc_h, x1_h, y1_h, x2_h, y2_h, thr_h,
                  osc_h, ox1_h, oy1_h, ox2_h, oy2_h, ofl_h, cnt_h,
                  s_sc, s_x1, s_y1, s_x2, s_y2, s_thr,
                  o_sc, o_x1, o_y1, o_x2, o_y2, o_fl,
                  c_stage, c_table, cf_stage, c_sh,
                  p_sc, p_x1, p_y1, p_x2, p_y2, p_fl,
                  r_ps, r_off, r_trun):
    cid = lax.axis_index("c")
    sid = lax.axis_index("s")

    @pl.when(cid == 0)
    def _work():
        wid = sid
        base = wid * _CH
        pltpu.sync_copy(sc_h.at[pl.ds(base, _CH)], s_sc)
        pltpu.sync_copy(x1_h.at[pl.ds(base, _CH)], s_x1)
        pltpu.sync_copy(y1_h.at[pl.ds(base, _CH)], s_y1)
        pltpu.sync_copy(x2_h.at[pl.ds(base, _CH)], s_x2)
        pltpu.sync_copy(y2_h.at[pl.ds(base, _CH)], s_y2)
        pltpu.sync_copy(thr_h.at[pl.ds(0, 16)], s_thr)
        thrv = s_thr[...]

        lane = lax.iota(jnp.int32, 16)
        zero16 = jnp.zeros((16,), jnp.int32)
        one16 = jnp.full((16,), 1, jnp.int32)

        def lgat(v, idx):
            return v.at[idx].get(mode="promise_in_bounds")

        def rsum(v):
            for k in (1, 2, 4, 8):
                v = v + lgat(v, (lane + k) & 15)
            return v

        def csum(v):
            for k in (1, 2, 4, 8):
                sh = lgat(v, jnp.maximum(lane - k, 0))
                v = v + jnp.where(lane >= k, sh, zero16)
            return v

        cnt1 = zero16
        cnt2 = zero16
        for i in range(_VC):
            v = s_sc[pl.ds(16 * i, 16)]
            cnt1 = cnt1 + jnp.where(v > thrv, one16, zero16)
            cnt2 = cnt2 + jnp.where(v == thrv, one16, zero16)
        c1s = rsum(cnt1)
        c2s = rsum(cnt2)

        widv = zero16 + wid
        c_stage[...] = jnp.where(lane == widv, c1s, zero16)
        pltpu.sync_copy(c_stage, c_sh.at[wid])
        c_stage[...] = jnp.where(lane == widv, c2s, zero16)
        pltpu.sync_copy(c_stage, c_sh.at[_NT + wid])
        plsc.subcore_barrier()
        pltpu.sync_copy(c_sh, c_table)

        cnt1s = zero16
        cnt2s = zero16
        for j in range(_NT):
            cnt1s = cnt1s + c_table[j]
            cnt2s = cnt2s + c_table[_NT + j]

        c1_tot = rsum(cnt1s)
        need2 = _PRE_TOPN - c1_tot
        tb = csum(cnt2s) - cnt2s
        kept2 = jnp.clip(need2 - tb, 0, cnt2s)
        kept = cnt1s + kept2
        padk = (kept + 7) & (-8)
        pg_ex = csum(padk) - padk
        total_padded = rsum(padk)
        my_tb = rsum(jnp.where(lane == widv, tb, zero16))
        my_pg = rsum(jnp.where(lane == widv, pg_ex, zero16))

        r_ps[...] = zero16
        r_off[...] = zero16
        r_trun[...] = zero16

        def step(i, carry):
            off_in = pl.multiple_of(16 * i, 8)
            sl = pl.ds(off_in, 16)
            xs = [s_sc[sl], s_x1[sl], s_y1[sl], s_x2[sl], s_y2[sl],
                  (base + 16 * i + lane).astype(jnp.float32)]
            v = xs[0]
            trun = r_trun[...]
            m1 = v > thrv
            m2 = v == thrv
            m2i = jnp.where(m2, one16, zero16)
            tex = trun + csum(m2i) - m2i
            keepm = m1 | (m2 & ((tex + my_tb) < need2))
            ki = jnp.where(keepm, one16, zero16)
            ks = rsum(ki)
            pcs = csum(ki)
            d = lane - (pcs - ki)
            kcur = ki
            for b in (1, 2, 4, 8):
                idxb = (lane + b) & 15
                d_in = lgat(d, idxb)
                k_in = lgat(kcur, idxb)
                take = (k_in == 1) & ((d_in & b) != 0)
                moved = (kcur == 1) & ((d & b) != 0)
                xs = [jnp.where(take, lgat(x, idxb), x) for x in xs]
                d = jnp.where(take, d_in - b, d)
                kcur = jnp.where(take, one16, jnp.where(moved, zero16, kcur))
            ps = r_ps[...]
            offv = r_off[...]
            idx1 = (lane - ps) & 15
            comb = [jnp.where(lane < ps, p, lgat(x, idx1))
                    for p, x in ((p_sc[...], xs[0]), (p_x1[...], xs[1]),
                                 (p_y1[...], xs[2]), (p_x2[...], xs[3]),
                                 (p_y2[...], xs[4]), (p_fl[...], xs[5]))]
            t = ps + ks
            te = t[0]

            @pl.when(te >= 16)
            def _emit():
                dst = pl.ds(pl.multiple_of(offv[0], 8), 16)
                o_sc[dst] = comb[0]
                o_x1[dst] = comb[1]
                o_y1[dst] = comb[2]
                o_x2[dst] = comb[3]
                o_y2[dst] = comb[4]
                o_fl[dst] = comb[5]

            full = t >= 16
            idx2 = (lane + (16 - ps)) & 15
            newp = [jnp.where(full, lgat(x, idx2), c)
                    for x, c in zip(xs, comb)]
            p_sc[...] = newp[0]
            p_x1[...] = newp[1]
            p_y1[...] = newp[2]
            p_x2[...] = newp[3]
            p_y2[...] = newp[4]
            p_fl[...] = newp[5]
            r_ps[...] = jnp.where(full, t - 16, t)
            r_off[...] = offv + jnp.where(full, jnp.full((16,), 16, jnp.int32), zero16)
            r_trun[...] = trun + rsum(m2i)
            return carry

        lax.fori_loop(0, _VC, step, jnp.int32(0))

        ps = r_ps[...]
        offv = r_off[...]
        dst = pl.ds(pl.multiple_of(offv[0], 8), 16)
        o_sc[dst] = jnp.where(lane < ps, p_sc[...], jnp.full((16,), _DUMP, jnp.float32))
        o_x1[dst] = p_x1[...]
        o_y1[dst] = p_y1[...]
        o_x2[dst] = p_x2[...]
        o_y2[dst] = p_y2[...]
        o_fl[dst] = p_fl[...]

        cntkv = offv + ps
        cntpadv = (cntkv + 7) & (-8)
        cntpad = cntpadv[0]
        my_pg_s = my_pg[0]

        srcoff = jnp.int32(0)
        for sz in (1024, 512, 256, 128, 64, 32, 16, 8):
            bit = (cntpad & sz) != 0
            so = srcoff

            @pl.when(bit)
            def _copy(sz=sz, so=so):
                so = pl.multiple_of(so, 8)
                dsto = pl.multiple_of(my_pg_s + so, 8)
                pltpu.sync_copy(o_sc.at[pl.ds(so, sz)], osc_h.at[pl.ds(dsto, sz)])
                pltpu.sync_copy(o_x1.at[pl.ds(so, sz)], ox1_h.at[pl.ds(dsto, sz)])
                pltpu.sync_copy(o_y1.at[pl.ds(so, sz)], oy1_h.at[pl.ds(dsto, sz)])
                pltpu.sync_copy(o_x2.at[pl.ds(so, sz)], ox2_h.at[pl.ds(dsto, sz)])
                pltpu.sync_copy(o_y2.at[pl.ds(so, sz)], oy2_h.at[pl.ds(dsto, sz)])
                pltpu.sync_copy(o_fl.at[pl.ds(so, sz)], ofl_h.at[pl.ds(dsto, sz)])

            srcoff = srcoff + (cntpad & sz)

        @pl.when(wid == 0)
        def _cnt():
            cf_stage[...] = total_padded.astype(jnp.float32)
            pltpu.sync_copy(cf_stage, cnt_h.at[pl.ds(0, 16)])


_SC_MESH = plsc.VectorSubcoreMesh(core_axis_name="c", subcore_axis_name="s")

_compact = functools.partial(
    pl.kernel,
    mesh=_SC_MESH,
    out_type=[jax.ShapeDtypeStruct((_KR * _KC,), jnp.float32)] * 6
    + [jax.ShapeDtypeStruct((1024,), jnp.float32)],
    scratch_types=[pltpu.VMEM((_CH,), jnp.float32)] * 5
    + [pltpu.VMEM((16,), jnp.float32)]
    + [pltpu.VMEM((_CH + 64,), jnp.float32)] * 6
    + [pltpu.VMEM((16,), jnp.int32),
       pltpu.VMEM((32, 16), jnp.int32),
       pltpu.VMEM((16,), jnp.float32),
       pltpu.VMEM_SHARED((32, 16), jnp.int32)]
    + [pltpu.VMEM((16,), jnp.float32)] * 6
    + [pltpu.VMEM((16,), jnp.int32)] * 3,
)(_compact_body)


def _nms_body(sc_ref, x1_ref, y1_ref, x2_ref, y2_ref, fl_ref, cnt_ref, out_ref):
    scv = sc_ref[...]
    x1 = x1_ref[...]
    y1 = y1_ref[...]
    x2 = x2_ref[...]
    y2 = y2_ref[...]
    flatf = fl_ref[...]
    areas = (x2 - x1 + 1.0) * (y2 - y1 + 1.0)

    cnt = cnt_ref[0:1, 0:1].astype(jnp.int32)
    ri = lax.broadcasted_iota(jnp.int32, (_KR, _KC), 0)
    ci = lax.broadcasted_iota(jnp.int32, (_KR, _KC), 1)
    slot = ri * _KC + ci
    guard = (slot < cnt) & (scv >= jnp.float32(-1e9))
    alive0 = jnp.where(guard, scv, jnp.float32(_DUMP))

    def rmax(v):
        return _tree(v, jnp.maximum, _KR)

    def rmin(v):
        return _tree(v, jnp.minimum, _KR)

    lane8 = lax.broadcasted_iota(jnp.int32, (1, 8), 1)

    def nms_body(i, alive):
        best = rmax(alive)
        validb = best > jnp.float32(-2e9)
        eq = alive == best
        fmin = rmin(jnp.where(eq, flatf, jnp.float32(3e38)))
        onehot = eq & (flatf == fmin)

        def pick(v):
            return rmax(jnp.where(onehot, v, jnp.float32(-3.4e38)))

        bx1 = pick(x1)
        by1 = pick(y1)
        bx2 = pick(x2)
        by2 = pick(y2)
        bar = pick(areas)

        xx1 = jnp.maximum(bx1, x1)
        yy1 = jnp.maximum(by1, y1)
        xx2 = jnp.minimum(bx2, x2)
        yy2 = jnp.minimum(by2, y2)
        iw = jnp.maximum(0.0, xx2 - xx1 + 1.0)
        ih = jnp.maximum(0.0, yy2 - yy1 + 1.0)
        inter = iw * ih
        iou = inter / (bar + areas - inter)
        alive = jnp.where(validb & (iou > _THRESH), jnp.float32(_DUMP), alive)

        vf = jnp.where(validb, jnp.float32(1.0), jnp.float32(0.0))
        vals = jnp.where(lane8 == 1, bx1,
               jnp.where(lane8 == 2, by1,
               jnp.where(lane8 == 3, bx2,
               jnp.where(lane8 == 4, by2, jnp.float32(0.0))))) * vf
        out_ref[pl.ds(i, 1), :] = vals
        return alive

    lax.fori_loop(0, _POST_TOPN, nms_body, alive0)


@functools.partial(jax.jit, static_argnames=())
def kernel(rpn_cls_score, rpn_bbox_pred):
    cls = rpn_cls_score.reshape(-1, 2)
    box = rpn_bbox_pred.reshape(-1, 4)

    def prep(v):
        return jnp.pad(v, (0, _PAD)).reshape(_R, _C)

    args = (prep(cls[:, 1]), prep(cls[:, 0]),
            prep(box[:, 0]), prep(box[:, 1]), prep(box[:, 2]), prep(box[:, 3]),
            jnp.asarray(_WA), jnp.asarray(_HA), jnp.asarray(_CXA), jnp.asarray(_CYA))

    sc, x1, y1, x2, y2, thr = pl.pallas_call(
        _decode_body,
        out_shape=[jax.ShapeDtypeStruct((_R, _C), jnp.float32)] * 5
        + [jax.ShapeDtypeStruct((8, 128), jnp.float32)],
    )(*args)

    csc, cx1, cy1, cx2, cy2, cfl, ccnt = _compact(
        sc.reshape(-1), x1.reshape(-1), y1.reshape(-1),
        x2.reshape(-1), y2.reshape(-1), thr.reshape(-1))

    out8 = pl.pallas_call(
        _nms_body,
        out_shape=jax.ShapeDtypeStruct((304, 8), jnp.float32),
    )(csc.reshape(_KR, _KC), cx1.reshape(_KR, _KC), cy1.reshape(_KR, _KC),
      cx2.reshape(_KR, _KC), cy2.reshape(_KR, _KC), cfl.reshape(_KR, _KC),
      ccnt.reshape(8, 128))
    return out8[:_POST_TOPN, :5]


# speedup vs baseline: 1.0097x; 1.0097x over previous
import functools

import jax
import jax.numpy as jnp
import numpy as np
from jax import lax
from jax.experimental import pallas as pl
from jax.experimental.pallas import tpu as pltpu
from jax.experimental.pallas import tpu_sc as plsc

_FEAT_STRIDE = 16
_H = 50
_W = 50
_A = 9
_N = _H * _W * _A
_R, _C = 176, 128
_PAD = _R * _C - _N
_PRE_TOPN = 6000
_POST_TOPN = 300
_THRESH = 0.7
_IMIN = -2147483648

_NT = 16
_CH = (_R * _C) // _NT
_CH2 = (_R * _C) // 32
_VC2 = _CH2 // 16
_VC = _CH // 16
_KR, _KC = 56, 128
_DUMP = -3e38


def _anchor_consts():
    scales = np.array([8.0, 16.0, 32.0])
    ratios = np.array([0.5, 1.0, 2.0])
    base = np.array([1.0, 1.0, 16.0, 16.0]) - 1
    w = base[2] - base[0] + 1
    h = base[3] - base[1] + 1
    x_ctr = base[0] + 0.5 * (w - 1)
    y_ctr = base[1] + 0.5 * (h - 1)
    size_ratios = w * h / ratios
    ws = np.round(np.sqrt(size_ratios))
    hs = np.round(ws * ratios)

    def _mk(ws_, hs_, xc, yc):
        ws_ = ws_[:, None]
        hs_ = hs_[:, None]
        return np.hstack([xc - 0.5 * (ws_ - 1), yc - 0.5 * (hs_ - 1),
                          xc + 0.5 * (ws_ - 1), yc + 0.5 * (hs_ - 1)])

    ratio_anchors = _mk(ws, hs, x_ctr, y_ctr)
    out = []
    for i in range(ratio_anchors.shape[0]):
        a = ratio_anchors[i]
        aw = a[2] - a[0] + 1
        ah = a[3] - a[1] + 1
        axc = a[0] + 0.5 * (aw - 1)
        ayc = a[1] + 0.5 * (ah - 1)
        out.append(_mk(aw * scales, ah * scales, axc, ayc))
    base9 = np.vstack(out).astype(np.float32)

    shift = np.arange(_W, dtype=np.float32) * np.float32(_FEAT_STRIDE)
    sx, sy = np.meshgrid(shift, shift)
    shifts = np.stack([sx.ravel(), sy.ravel(), sx.ravel(), sy.ravel()], axis=1)
    anchors = (shifts[:, None, :].astype(np.float32)
               + base9[None, :, :]).reshape(-1, 4).astype(np.float32)
    x1, y1, x2, y2 = anchors[:, 0], anchors[:, 1], anchors[:, 2], anchors[:, 3]
    wa = x2 - x1 + np.float32(1.0)
    ha = y2 - y1 + np.float32(1.0)
    cxa = x1 + np.float32(0.5) * wa
    cya = y1 + np.float32(0.5) * ha

    def _pad(v):
        return np.pad(v, (0, _PAD)).reshape(_R, _C)

    return _pad(wa), _pad(ha), _pad(cxa), _pad(cya)


_WA, _HA, _CXA, _CYA = _anchor_consts()


def _tree(v, op, rows):
    chunks = [v[8 * i:8 * (i + 1)] for i in range(rows // 8)]
    while len(chunks) > 1:
        nxt = [op(chunks[2 * j], chunks[2 * j + 1])
               for j in range(len(chunks) // 2)]
        if len(chunks) % 2:
            nxt.append(chunks[-1])
        chunks = nxt
    r = chunks[0]
    if op is jnp.minimum:
        return jnp.min(jnp.min(r, axis=0, keepdims=True), axis=1, keepdims=True)
    if op is jnp.add:
        return jnp.sum(jnp.sum(r, axis=0, keepdims=True), axis=1, keepdims=True)
    return jnp.max(jnp.max(r, axis=0, keepdims=True), axis=1, keepdims=True)


def _decode_body(fg_ref, bg_ref, dx_ref, dy_ref, dw_ref, dh_ref,
                 wa_ref, ha_ref, cx_ref, cy_ref,
                 sc_ref, x1_ref, y1_ref, x2_ref, y2_ref, thr_ref):
    fg = fg_ref[...]
    bg = bg_ref[...]
    wa = wa_ref[...]
    ha = ha_ref[...]

    m = jnp.maximum(fg, bg)
    ef = jnp.exp(fg - m)
    eb = jnp.exp(bg - m)
    sc = ef / (eb + ef)

    pcx = dx_ref[...] * wa + cx_ref[...]
    pcy = dy_ref[...] * ha + cy_ref[...]
    pw = jnp.exp(dw_ref[...]) * wa
    ph = jnp.exp(dh_ref[...]) * ha
    x1 = jnp.clip(pcx - 0.5 * pw, 0.0, 799.0)
    y1 = jnp.clip(pcy - 0.5 * ph, 0.0, 799.0)
    x2 = jnp.clip(pcx + 0.5 * pw, 0.0, 799.0)
    y2 = jnp.clip(pcy + 0.5 * ph, 0.0, 799.0)

    ws = x2 - x1 + 1.0
    hs = y2 - y1 + 1.0
    ok = (ws >= 16.0) & (hs >= 16.0)
    sc = jnp.where(ok, sc, jnp.float32(-1e9))

    ri = lax.broadcasted_iota(jnp.int32, (_R, _C), 0)
    ci = lax.broadcasted_iota(jnp.int32, (_R, _C), 1)
    flat = ri * _C + ci
    sc = jnp.where(flat < _N, sc, jnp.float32(-2e9))

    key = lax.bitcast_convert_type(sc, jnp.int32)
    key = key ^ ((key >> 31) & jnp.int32(0x7FFFFFFF))

    def bs(_, lh):
        lo, hi = lh
        mid = (lo >> 1) + (hi >> 1) + ((lo | hi) & 1)
        cnt = _tree((key >= mid).astype(jnp.int32), jnp.add, _R)
        p = cnt >= _PRE_TOPN
        return (jnp.where(p, mid, lo), jnp.where(p, hi, mid - 1))

    thr, _ = lax.fori_loop(
        0, 32, bs,
        (jnp.full((1, 1), _IMIN, jnp.int32),
         jnp.full((1, 1), 2147483647, jnp.int32)))
    ty = jnp.where(thr >= 0, thr, thr ^ jnp.int32(0x7FFFFFFF))
    thrf = lax.bitcast_convert_type(ty, jnp.float32)

    sc_ref[...] = sc
    x1_ref[...] = x1
    y1_ref[...] = y1
    x2_ref[...] = x2
    y2_ref[...] = y2
    thr_ref[...] = jnp.broadcast_to(thrf, (8, 128))


def _compact_body(sc_h, x1_h, y1_h, x2_h, y2_h, thr_h,
                  osc_h, ox1_h, oy1_h, ox2_h, oy2_h, ofl_h, cnt_h,
                  s_sc, s_x1, s_y1, s_x2, s_y2, s_thr,
                  o_sc, o_x1, o_y1, o_x2, o_y2, o_fl,
                  c_stage, c_table, cf_stage, c_sh,
                  p_sc, p_x1, p_y1, p_x2, p_y2, p_fl,
                  r_ps, r_off, r_trun):
    cid = lax.axis_index("c")
    sid = lax.axis_index("s")

    w = 2 * sid + cid
    wsib = 2 * sid + (1 - cid)
    g = sid // 8
    s7 = sid % 8
    base = pl.multiple_of(w * _CH2, 8)
    bsib = pl.multiple_of(wsib * _CH2, 8)

    pltpu.sync_copy(sc_h.at[pl.ds(base, _CH2)], s_sc.at[pl.ds(0, _CH2)])
    pltpu.sync_copy(sc_h.at[pl.ds(bsib, _CH2)], s_sc.at[pl.ds(_CH2, _CH2)])
    pltpu.sync_copy(x1_h.at[pl.ds(base, _CH2)], s_x1)
    pltpu.sync_copy(y1_h.at[pl.ds(base, _CH2)], s_y1)
    pltpu.sync_copy(x2_h.at[pl.ds(base, _CH2)], s_x2)
    pltpu.sync_copy(y2_h.at[pl.ds(base, _CH2)], s_y2)
    pltpu.sync_copy(thr_h.at[pl.ds(0, 16)], s_thr)
    thrv = s_thr[...]

    lane = lax.iota(jnp.int32, 16)
    zero16 = jnp.zeros((16,), jnp.int32)
    one16 = jnp.full((16,), 1, jnp.int32)

    def lgat(v, idx):
        return v.at[idx].get(mode="promise_in_bounds")

    def rsum(v):
        for k in (1, 2, 4, 8):
            v = v + lgat(v, (lane + k) & 15)
        return v

    def csum(v):
        for k in (1, 2, 4, 8):
            sh = lgat(v, jnp.maximum(lane - k, 0))
            v = v + jnp.where(lane >= k, sh, zero16)
        return v

    cnt1a = zero16
    cnt2a = zero16
    cnt1b = zero16
    cnt2b = zero16
    for i in range(_VC2):
        v = s_sc[pl.ds(16 * i, 16)]
        cnt1a = cnt1a + jnp.where(v > thrv, one16, zero16)
        cnt2a = cnt2a + jnp.where(v == thrv, one16, zero16)
    for i in range(_VC2, 2 * _VC2):
        v = s_sc[pl.ds(16 * i, 16)]
        cnt1b = cnt1b + jnp.where(v > thrv, one16, zero16)
        cnt2b = cnt2b + jnp.where(v == thrv, one16, zero16)
    c1self = rsum(cnt1a)
    c2self = rsum(cnt2a)
    c1sib = rsum(cnt1b)
    c2sib = rsum(cnt2b)

    lself = zero16 + (2 * s7 + cid)
    lsib = zero16 + (2 * s7 + (1 - cid))
    r1 = g * 8 + s7
    c_stage[...] = jnp.where(lane == lself, c1self,
                             jnp.where(lane == lsib, c1sib, zero16))
    pltpu.sync_copy(c_stage, c_sh.at[r1])
    c_stage[...] = jnp.where(lane == lself, c2self,
                             jnp.where(lane == lsib, c2sib, zero16))
    pltpu.sync_copy(c_stage, c_sh.at[16 + r1])
    plsc.subcore_barrier()
    pltpu.sync_copy(c_sh, c_table)

    c1A = zero16
    c1B = zero16
    c2A = zero16
    c2B = zero16
    for j in range(8):
        c1A = c1A + c_table[j]
        c1B = c1B + c_table[8 + j]
        c2A = c2A + c_table[16 + j]
        c2B = c2B + c_table[24 + j]

    c1_tot = rsum(c1A) + rsum(c1B)
    need2 = _PRE_TOPN - c1_tot
    tbA = csum(c2A) - c2A
    tbB = csum(c2B) - c2B + rsum(c2A)
    kA = c1A + jnp.clip(need2 - tbA, 0, c2A)
    kB = c1B + jnp.clip(need2 - tbB, 0, c2B)
    pdA = (kA + 7) & (-8)
    pdB = (kB + 7) & (-8)
    pgA = csum(pdA) - pdA
    pgB = csum(pdB) - pdB + rsum(pdA)
    total_padded = rsum(pdA) + rsum(pdB)
    gv = zero16 + g
    wl = zero16 + (w - 16 * g)
    pick_g = tbA * (1 - gv) + tbB * gv
    my_tb = rsum(jnp.where(lane == wl, pick_g, zero16))
    pick_p = pgA * (1 - gv) + pgB * gv
    my_pg = rsum(jnp.where(lane == wl, pick_p, zero16))

    r_ps[...] = zero16
    r_off[...] = zero16
    r_trun[...] = zero16

    def step(i, carry):
        off_in = pl.multiple_of(16 * i, 8)
        sl = pl.ds(off_in, 16)
        xs = [s_sc[sl], s_x1[sl], s_y1[sl], s_x2[sl], s_y2[sl],
              (base + 16 * i + lane).astype(jnp.float32)]
        v = xs[0]
        trun = r_trun[...]
        m1 = v > thrv
        m2 = v == thrv
        m2i = jnp.where(m2, one16, zero16)
        tex = trun + csum(m2i) - m2i
        keepm = m1 | (m2 & ((tex + my_tb) < need2))
        ki = jnp.where(keepm, one16, zero16)
        ks = rsum(ki)
        pcs = csum(ki)
        d = lane - (pcs - ki)
        kcur = ki
        srci = lane
        for b in (1, 2, 4, 8):
            idxb = (lane + b) & 15
            d_in = lgat(d, idxb)
            k_in = lgat(kcur, idxb)
            take = (k_in == 1) & ((d_in & b) != 0)
            moved = (kcur == 1) & ((d & b) != 0)
            srci = jnp.where(take, lgat(srci, idxb), srci)
            d = jnp.where(take, d_in - b, d)
            kcur = jnp.where(take, one16, jnp.where(moved, zero16, kcur))
        ps = r_ps[...]
        offv = r_off[...]
        idx1 = lgat(srci, (lane - ps) & 15)
        idx2 = lgat(srci, (lane + (16 - ps)) & 15)
        comb = [jnp.where(lane < ps, p, lgat(x, idx1))
                for p, x in ((p_sc[...], xs[0]), (p_x1[...], xs[1]),
                             (p_y1[...], xs[2]), (p_x2[...], xs[3]),
                             (p_y2[...], xs[4]), (p_fl[...], xs[5]))]
        t = ps + ks
        te = t[0]

        @pl.when(te >= 16)
        def _emit():
            dst = pl.ds(pl.multiple_of(offv[0], 8), 16)
            o_sc[dst] = comb[0]
            o_x1[dst] = comb[1]
            o_y1[dst] = comb[2]
            o_x2[dst] = comb[3]
            o_y2[dst] = comb[4]
            o_fl[dst] = comb[5]

        full = t >= 16
        newp = [jnp.where(full, lgat(x, idx2), c)
                for x, c in zip(xs, comb)]
        p_sc[...] = newp[0]
        p_x1[...] = newp[1]
        p_y1[...] = newp[2]
        p_x2[...] = newp[3]
        p_y2[...] = newp[4]
        p_fl[...] = newp[5]
        r_ps[...] = jnp.where(full, t - 16, t)
        r_off[...] = offv + jnp.where(full, jnp.full((16,), 16, jnp.int32), zero16)
        r_trun[...] = trun + rsum(m2i)
        return carry

    lax.fori_loop(0, _VC2, step, jnp.int32(0))

    ps = r_ps[...]
    offv = r_off[...]
    dst = pl.ds(pl.multiple_of(offv[0], 8), 16)
    o_sc[dst] = jnp.where(lane < ps, p_sc[...], jnp.full((16,), _DUMP, jnp.float32))
    o_x1[dst] = p_x1[...]
    o_y1[dst] = p_y1[...]
    o_x2[dst] = p_x2[...]
    o_y2[dst] = p_y2[...]
    o_fl[dst] = p_fl[...]

    cntkv = offv + ps
    cntpadv = (cntkv + 7) & (-8)
    cntpad = cntpadv[0]
    my_pg_s = my_pg[0]

    srcoff = jnp.int32(0)
    for sz in (512, 256, 128, 64, 32, 16, 8):
        bit = (cntpad & sz) != 0
        so = srcoff

        @pl.when(bit)
        def _copy(sz=sz, so=so):
            so = pl.multiple_of(so, 8)
            dsto = pl.multiple_of(my_pg_s + so, 8)
            pltpu.sync_copy(o_sc.at[pl.ds(so, sz)], osc_h.at[pl.ds(dsto, sz)])
            pltpu.sync_copy(o_x1.at[pl.ds(so, sz)], ox1_h.at[pl.ds(dsto, sz)])
            pltpu.sync_copy(o_y1.at[pl.ds(so, sz)], oy1_h.at[pl.ds(dsto, sz)])
            pltpu.sync_copy(o_x2.at[pl.ds(so, sz)], ox2_h.at[pl.ds(dsto, sz)])
            pltpu.sync_copy(o_y2.at[pl.ds(so, sz)], oy2_h.at[pl.ds(dsto, sz)])
            pltpu.sync_copy(o_fl.at[pl.ds(so, sz)], ofl_h.at[pl.ds(dsto, sz)])

        srcoff = srcoff + (cntpad & sz)

    @pl.when((cid == 0) & (sid == 0))
    def _cnt():
        cf_stage[...] = total_padded.astype(jnp.float32)
        pltpu.sync_copy(cf_stage, cnt_h.at[pl.ds(0, 16)])


_SC_MESH = plsc.VectorSubcoreMesh(core_axis_name="c", subcore_axis_name="s")

_compact = functools.partial(
    pl.kernel,
    mesh=_SC_MESH,
    out_type=[jax.ShapeDtypeStruct((_KR * _KC,), jnp.float32)] * 6
    + [jax.ShapeDtypeStruct((1024,), jnp.float32)],
    scratch_types=[pltpu.VMEM((2 * _CH2,), jnp.float32)]
    + [pltpu.VMEM((_CH2,), jnp.float32)] * 4
    + [pltpu.VMEM((16,), jnp.float32)]
    + [pltpu.VMEM((_CH2 + 64,), jnp.float32)] * 6
    + [pltpu.VMEM((16,), jnp.int32),
       pltpu.VMEM((32, 16), jnp.int32),
       pltpu.VMEM((16,), jnp.float32),
       pltpu.VMEM_SHARED((32, 16), jnp.int32)]
    + [pltpu.VMEM((16,), jnp.float32)] * 6
    + [pltpu.VMEM((16,), jnp.int32)] * 3,
)(_compact_body)


def _nms_body(sc_ref, x1_ref, y1_ref, x2_ref, y2_ref, fl_ref, cnt_ref, out_ref):
    scv = sc_ref[...]
    x1 = x1_ref[...]
    y1 = y1_ref[...]
    x2 = x2_ref[...]
    y2 = y2_ref[...]
    flatf = fl_ref[...]
    areas = (x2 - x1 + 1.0) * (y2 - y1 + 1.0)

    cnt = cnt_ref[0:1, 0:1].astype(jnp.int32)
    ri = lax.broadcasted_iota(jnp.int32, (_KR, _KC), 0)
    ci = lax.broadcasted_iota(jnp.int32, (_KR, _KC), 1)
    slot = ri * _KC + ci
    guard = (slot < cnt) & (scv >= jnp.float32(-1e9))
    alive0 = jnp.where(guard, scv, jnp.float32(_DUMP))

    def rmax(v):
        return _tree(v, jnp.maximum, _KR)

    def rmin(v):
        return _tree(v, jnp.minimum, _KR)

    lane8 = lax.broadcasted_iota(jnp.int32, (1, 8), 1)

    def nms_body(i, alive):
        best = rmax(alive)
        validb = best > jnp.float32(-2e9)
        eq = alive == best
        fmin = rmin(jnp.where(eq, flatf, jnp.float32(3e38)))
        onehot = eq & (flatf == fmin)

        def pick(v):
            return rmax(jnp.where(onehot, v, jnp.float32(-3.4e38)))

        bx1 = pick(x1)
        by1 = pick(y1)
        bx2 = pick(x2)
        by2 = pick(y2)
        bar = pick(areas)

        xx1 = jnp.maximum(bx1, x1)
        yy1 = jnp.maximum(by1, y1)
        xx2 = jnp.minimum(bx2, x2)
        yy2 = jnp.minimum(by2, y2)
        iw = jnp.maximum(0.0, xx2 - xx1 + 1.0)
        ih = jnp.maximum(0.0, yy2 - yy1 + 1.0)
        inter = iw * ih
        iou = inter / (bar + areas - inter)
        alive = jnp.where(validb & (iou > _THRESH), jnp.float32(_DUMP), alive)

        vf = jnp.where(validb, jnp.float32(1.0), jnp.float32(0.0))
        vals = jnp.where(lane8 == 1, bx1,
               jnp.where(lane8 == 2, by1,
               jnp.where(lane8 == 3, bx2,
               jnp.where(lane8 == 4, by2, jnp.float32(0.0))))) * vf
        out_ref[pl.ds(i, 1), :] = vals
        return alive

    lax.fori_loop(0, _POST_TOPN, nms_body, alive0)


@functools.partial(jax.jit, static_argnames=())
def kernel(rpn_cls_score, rpn_bbox_pred):
    cls = rpn_cls_score.reshape(-1, 2)
    box = rpn_bbox_pred.reshape(-1, 4)

    def prep(v):
        return jnp.pad(v, (0, _PAD)).reshape(_R, _C)

    args = (prep(cls[:, 1]), prep(cls[:, 0]),
            prep(box[:, 0]), prep(box[:, 1]), prep(box[:, 2]), prep(box[:, 3]),
            jnp.asarray(_WA), jnp.asarray(_HA), jnp.asarray(_CXA), jnp.asarray(_CYA))

    sc, x1, y1, x2, y2, thr = pl.pallas_call(
        _decode_body,
        out_shape=[jax.ShapeDtypeStruct((_R, _C), jnp.float32)] * 5
        + [jax.ShapeDtypeStruct((8, 128), jnp.float32)],
    )(*args)

    csc, cx1, cy1, cx2, cy2, cfl, ccnt = _compact(
        sc.reshape(-1), x1.reshape(-1), y1.reshape(-1),
        x2.reshape(-1), y2.reshape(-1), thr.reshape(-1))

    out8 = pl.pallas_call(
        _nms_body,
        out_shape=jax.ShapeDtypeStruct((304, 8), jnp.float32),
    )(csc.reshape(_KR, _KC), cx1.reshape(_KR, _KC), cy1.reshape(_KR, _KC),
      cx2.reshape(_KR, _KC), cy2.reshape(_KR, _KC), cfl.reshape(_KR, _KC),
      ccnt.reshape(8, 128))
    return out8[:_POST_TOPN, :5]
